# Initial kernel scaffold; baseline (speedup 1.0000x reference)
import jax
import jax.numpy as jnp
from jax.experimental import pallas as pl


def kernel(x, edge_index, W1, b1, W2, b2, W3, b3, W4, b4, W5, b5, W6, b6, Wp, bp):
    raise NotImplementedError("write your pallas kernel here")


# speedup vs baseline: 5.9561x; 5.9561x over previous
import functools

import jax
import jax.numpy as jnp
from jax import lax
from jax.experimental import pallas as pl
from jax.experimental.pallas import tpu as pltpu
from jax.experimental.pallas import tpu_sc as plsc

N = 10000
N_PAD = 10240
E = 320000
W128 = 128
NS = 16
NC = 2
STRIPE = N_PAD // NS
CHUNK = 128
SEC = 8
CHUNKS_COL = 160
CHUNKS_EDGE = 80
SECS_COL = CHUNKS_COL // SEC
SECS_EDGE = CHUNKS_EDGE // SEC
E_PAD = NS * CHUNKS_COL * CHUNK
DST_SINK = N

_mesh = plsc.VectorSubcoreMesh(core_axis_name="c", subcore_axis_name="s")


def _fill_ones(buf, rows):
    @pl.loop(0, rows)
    def _fill(i):
        for k in range(8):
            buf[i, pl.ds(16 * k, 16)] = jnp.full((16,), 1.0, jnp.float32)


@functools.partial(
    pl.kernel,
    out_type=(jax.ShapeDtypeStruct((N_PAD, W128), jnp.float32),
              jax.ShapeDtypeStruct((N_PAD, W128), jnp.float32)),
    mesh=_mesh,
    scratch_types=[
        pltpu.VMEM((SEC, CHUNK), jnp.int32),
        pltpu.VMEM((CHUNK, W128), jnp.float32),
        pltpu.VMEM_SHARED((N_PAD, W128), jnp.float32),
    ],
)
def _deg_kernel(dst3, zeros, p0, p1, dstbuf, ones_buf, acc):
    cid = lax.axis_index("c")
    sid = lax.axis_index("s")
    r0 = sid * STRIPE
    wid = cid * NS + sid

    _fill_ones(ones_buf, CHUNK)

    @pl.when(cid == 0)
    def _init0():
        for k in range(STRIPE // CHUNK):
            pltpu.sync_copy(ones_buf, acc.at[pl.ds(r0 + k * CHUNK, CHUNK)])

    @pl.when(cid == 1)
    def _init1():
        pltpu.sync_copy(zeros, acc.at[pl.ds(r0, STRIPE)])

    plsc.subcore_barrier()

    @pl.loop(0, SECS_EDGE)
    def _sec(s):
        pltpu.sync_copy(dst3.at[wid].at[pl.ds(s * SEC, SEC)], dstbuf)
        for k in range(SEC):
            pltpu.sync_copy(ones_buf, acc.at[dstbuf.at[k]], add=True)

    plsc.subcore_barrier()

    @pl.when(cid == 0)
    def _out0():
        pltpu.sync_copy(acc.at[pl.ds(r0, STRIPE)], p0.at[pl.ds(r0, STRIPE)])

    @pl.when(cid == 1)
    def _out1():
        pltpu.sync_copy(acc.at[pl.ds(r0, STRIPE)], p1.at[pl.ds(r0, STRIPE)])


@functools.partial(
    pl.kernel,
    out_type=(jax.ShapeDtypeStruct((N_PAD, W128), jnp.float32),
              jax.ShapeDtypeStruct((N_PAD, W128), jnp.float32)),
    mesh=_mesh,
    scratch_types=[
        pltpu.VMEM((SEC, CHUNK), jnp.int32),
        pltpu.VMEM((SEC, CHUNK), jnp.int32),
        pltpu.VMEM((CHUNK, W128), jnp.float32),
        pltpu.VMEM_SHARED((N_PAD, W128), jnp.float32),
    ],
)
def _agg_edge(t, src3, dst3, zeros, p0, p1, srcbuf, dstbuf, rows, acc):
    cid = lax.axis_index("c")
    sid = lax.axis_index("s")
    r0 = sid * STRIPE
    wid = cid * NS + sid

    @pl.when(cid == 0)
    def _init0():
        pltpu.sync_copy(t.at[pl.ds(r0, STRIPE)], acc.at[pl.ds(r0, STRIPE)])

    @pl.when(cid == 1)
    def _init1():
        pltpu.sync_copy(zeros, acc.at[pl.ds(r0, STRIPE)])

    plsc.subcore_barrier()

    @pl.loop(0, SECS_EDGE)
    def _sec(s):
        pltpu.sync_copy(src3.at[wid].at[pl.ds(s * SEC, SEC)], srcbuf)
        pltpu.sync_copy(dst3.at[wid].at[pl.ds(s * SEC, SEC)], dstbuf)
        for k in range(SEC):
            pltpu.sync_copy(t.at[srcbuf.at[k]], rows)
            pltpu.sync_copy(rows, acc.at[dstbuf.at[k]], add=True)

    plsc.subcore_barrier()

    @pl.when(cid == 0)
    def _out0():
        pltpu.sync_copy(acc.at[pl.ds(r0, STRIPE)], p0.at[pl.ds(r0, STRIPE)])

    @pl.when(cid == 1)
    def _out1():
        pltpu.sync_copy(acc.at[pl.ds(r0, STRIPE)], p1.at[pl.ds(r0, STRIPE)])


@functools.partial(
    pl.kernel,
    out_type=(jax.ShapeDtypeStruct((N_PAD, W128), jnp.float32),
              jax.ShapeDtypeStruct((N_PAD, W128), jnp.float32)),
    mesh=_mesh,
    scratch_types=[
        pltpu.VMEM((SEC, CHUNK), jnp.int32),
        pltpu.VMEM((SEC, CHUNK), jnp.int32),
        pltpu.VMEM((CHUNK, W128), jnp.float32),
        pltpu.VMEM_SHARED((N_PAD, W128), jnp.float32),
    ],
)
def _agg_col(t_a, t_b, src3, dst3, s_a, s_b, srcbuf, dstbuf, rows, acc):
    cid = lax.axis_index("c")
    sid = lax.axis_index("s")
    r0 = sid * STRIPE

    @pl.when(cid == 0)
    def _init_a():
        pltpu.sync_copy(t_a.at[pl.ds(r0, STRIPE)], acc.at[pl.ds(r0, STRIPE)])

    @pl.when(cid == 1)
    def _init_b():
        pltpu.sync_copy(t_b.at[pl.ds(r0, STRIPE)], acc.at[pl.ds(r0, STRIPE)])

    plsc.subcore_barrier()

    for c_val, tref in ((0, t_a), (1, t_b)):
        @pl.when(cid == c_val)
        def _scatter():
            @pl.loop(0, SECS_COL)
            def _sec(s):
                pltpu.sync_copy(src3.at[sid].at[pl.ds(s * SEC, SEC)], srcbuf)
                pltpu.sync_copy(dst3.at[sid].at[pl.ds(s * SEC, SEC)], dstbuf)
                for k in range(SEC):
                    pltpu.sync_copy(tref.at[srcbuf.at[k]], rows)
                    pltpu.sync_copy(rows, acc.at[dstbuf.at[k]], add=True)

    plsc.subcore_barrier()

    @pl.when(cid == 0)
    def _out_a():
        pltpu.sync_copy(acc.at[pl.ds(r0, STRIPE)], s_a.at[pl.ds(r0, STRIPE)])

    @pl.when(cid == 1)
    def _out_b():
        pltpu.sync_copy(acc.at[pl.ds(r0, STRIPE)], s_b.at[pl.ds(r0, STRIPE)])


_R = 1024
_GRID = (N_PAD // _R,)
_HI = lax.Precision.HIGHEST


def _rows_spec(w):
    return pl.BlockSpec((_R, w), lambda i: (i, 0))


def _full_spec(a, b):
    return pl.BlockSpec((a, b), lambda i: (0, 0))


def _leaky(y):
    return jnp.where(y >= 0, y, 0.01 * y)


def _shapes(*ws):
    return tuple(jax.ShapeDtypeStruct((N_PAD, w), jnp.float32) for w in ws)


def _tc0_body(degp0, degp1, x, dinv16, t1):
    dv = lax.rsqrt(degp0[:, 0:1] + degp1[:, 0:1])
    dinv16[...] = jnp.broadcast_to(dv, (_R, 16))
    t1[...] = x[...] * dv


def _tc0(degp0, degp1, x):
    return pl.pallas_call(
        _tc0_body,
        grid=_GRID,
        in_specs=[_rows_spec(W128), _rows_spec(W128), _rows_spec(128)],
        out_specs=(_rows_spec(16), _rows_spec(128)),
        out_shape=(jax.ShapeDtypeStruct((N_PAD, 16), jnp.float32),) + _shapes(128),
    )(degp0, degp1, x)


def _tc_aa_body(dinv16, p0, p1, W, b, t_next):
    dv = dinv16[:, 0:1]
    s = p0[...] + p1[...]
    y = jnp.dot(s * dv, W[...], precision=_HI,
                preferred_element_type=jnp.float32) + b[...]
    t_next[...] = _leaky(y) * dv


def _tc_aa(dinv16, p0, p1, W, b):
    return pl.pallas_call(
        _tc_aa_body,
        grid=_GRID,
        in_specs=[_rows_spec(16), _rows_spec(128), _rows_spec(128),
                  _full_spec(128, 128), _full_spec(1, 128)],
        out_specs=_rows_spec(128),
        out_shape=_shapes(128)[0],
    )(dinv16, p0, p1, W, b)


def _tc_am_body(dinv16, p0, p1, W1, b1, W2, ta, tb):
    dv = dinv16[:, 0:1]
    s = p0[...] + p1[...]
    y = jnp.dot(s * dv, W1[...], precision=_HI,
                preferred_element_type=jnp.float32) + b1[...]
    t = jnp.dot(_leaky(y), W2[...], precision=_HI,
                preferred_element_type=jnp.float32) * dv
    ta[...] = t[:, :128]
    tb[...] = t[:, 128:]


def _tc_am(dinv16, p0, p1, W1, b1, W2):
    return pl.pallas_call(
        _tc_am_body,
        grid=_GRID,
        in_specs=[_rows_spec(16), _rows_spec(128), _rows_spec(128),
                  _full_spec(128, 256), _full_spec(1, 256), _full_spec(256, 256)],
        out_specs=(_rows_spec(128), _rows_spec(128)),
        out_shape=_shapes(128, 128),
    )(dinv16, p0, p1, W1, b1, W2)


def _tc_mm_body(dinv16, sa, sb, b, W, t_next):
    dv = dinv16[:, 0:1]
    s = jnp.concatenate([sa[...], sb[...]], axis=1)
    a = _leaky(s * dv + b[...])
    t_next[...] = jnp.dot(a, W[...], precision=_HI,
                          preferred_element_type=jnp.float32) * dv


def _tc_mm(dinv16, sa, sb, b, W):
    return pl.pallas_call(
        _tc_mm_body,
        grid=_GRID,
        in_specs=[_rows_spec(16), _rows_spec(128), _rows_spec(128),
                  _full_spec(1, 256), _full_spec(256, 128)],
        out_specs=_rows_spec(128),
        out_shape=_shapes(128)[0],
    )(dinv16, sa, sb, b, W)


def _tc_ma_body(dinv16, p0, p1, b, t_next):
    dv = dinv16[:, 0:1]
    s = p0[...] + p1[...]
    t_next[...] = _leaky(s * dv + b[...]) * dv


def _tc_ma(dinv16, p0, p1, b):
    return pl.pallas_call(
        _tc_ma_body,
        grid=_GRID,
        in_specs=[_rows_spec(16), _rows_spec(128), _rows_spec(128),
                  _full_spec(1, 128)],
        out_specs=_rows_spec(128),
        out_shape=_shapes(128)[0],
    )(dinv16, p0, p1, b)


def _tc_final_body(dinv16, p0, p1, W6, b6, Wp, bp, out):
    dv = dinv16[:, 0:1]
    s = p0[...] + p1[...]
    y = jnp.dot(s * dv, W6[...], precision=_HI,
                preferred_element_type=jnp.float32) + b6[...]
    out[...] = jnp.dot(_leaky(y), Wp[...], precision=_HI,
                       preferred_element_type=jnp.float32) + bp[...]


def _tc_final(dinv16, p0, p1, W6, b6, Wp, bp):
    return pl.pallas_call(
        _tc_final_body,
        grid=_GRID,
        in_specs=[_rows_spec(16), _rows_spec(128), _rows_spec(128),
                  _full_spec(128, 64), _full_spec(1, 64),
                  _full_spec(64, 40), _full_spec(1, 40)],
        out_specs=_rows_spec(40),
        out_shape=jax.ShapeDtypeStruct((N_PAD, 40), jnp.float32),
    )(dinv16, p0, p1, W6, b6, Wp, bp)


def _pad_reshape(a, parts, chunks, fill):
    pad = parts * chunks * CHUNK - E
    return jnp.concatenate(
        [a, jnp.full((pad,), fill, jnp.int32)]).reshape(parts, chunks, CHUNK)


def kernel(x, edge_index, W1, b1, W2, b2, W3, b3, W4, b4, W5, b5, W6, b6, Wp, bp):
    src = edge_index[0].astype(jnp.int32)
    dst = edge_index[1].astype(jnp.int32)
    src3c = _pad_reshape(src, NS, CHUNKS_COL, 0)
    dst3c = _pad_reshape(dst, NS, CHUNKS_COL, DST_SINK)
    src3e = _pad_reshape(src, NS * NC, CHUNKS_EDGE, 0)
    dst3e = _pad_reshape(dst, NS * NC, CHUNKS_EDGE, DST_SINK)

    xp = jnp.pad(x, ((0, N_PAD - N), (0, 0)))
    zeros = jnp.zeros((STRIPE, W128), jnp.float32)

    b1r, b2r, b3r, b4r, b5r, b6r, bpr = (
        b.reshape(1, -1) for b in (b1, b2, b3, b4, b5, b6, bp))

    dg0, dg1 = _deg_kernel(dst3e, zeros)
    dinv16, t1 = _tc0(dg0, dg1, xp)
    p0, p1 = _agg_edge(t1, src3e, dst3e, zeros)
    t2 = _tc_aa(dinv16, p0, p1, W1, b1r)
    p0, p1 = _agg_edge(t2, src3e, dst3e, zeros)
    t3 = _tc_aa(dinv16, p0, p1, W2, b2r)
    p0, p1 = _agg_edge(t3, src3e, dst3e, zeros)
    t4a, t4b = _tc_am(dinv16, p0, p1, W3, b3r, W4)
    s4a, s4b = _agg_col(t4a, t4b, src3c, dst3c)
    t5 = _tc_mm(dinv16, s4a, s4b, b4r, W5)
    p0, p1 = _agg_edge(t5, src3e, dst3e, zeros)
    t6 = _tc_ma(dinv16, p0, p1, b5r)
    p0, p1 = _agg_edge(t6, src3e, dst3e, zeros)
    out = _tc_final(dinv16, p0, p1, W6, b6r, Wp, bpr)
    return out[:N]
